# baseline (speedup 1.0000x reference)
import jax
import jax.numpy as jnp
from jax import lax
from jax.experimental import pallas as pl
from jax.experimental.pallas import tpu as pltpu
from jax.experimental.pallas import tpu_sc as plsc

D = 64
B = 16384
L = 16
NC, NS = 2, 16
NW = NC * NS
BPW = B // NW
K = 16


def _perm(x, idx):
    dnums = lax.GatherDimensionNumbers(
        offset_dims=(), collapsed_slice_dims=(0,), start_index_map=(0,))
    return lax.gather(x, idx[:, None], dnums, slice_sizes=(1,),
                      mode=lax.GatherScatterMode.PROMISE_IN_BOUNDS)


def _sc_body(user_idx, cat_idx, user_table, cat_table, w_flat, b_vec,
             out_hbm, uidx_v, cidx_v, u_rows, c_rows, out_v, w_v, b_v,
             sem_u, sem_c):
    wid = lax.axis_index("s") * NC + lax.axis_index("c")
    base = wid * BPW

    pltpu.sync_copy(user_idx.at[pl.ds(base, BPW)], uidx_v)
    pltpu.sync_copy(cat_idx.at[pl.ds(base, BPW)], cidx_v)
    pltpu.sync_copy(w_flat, w_v)
    pltpu.sync_copy(b_vec, b_v)

    def drain_u():
        pltpu.make_async_copy(user_table.at[0], u_rows.at[0, pl.ds(0, D)],
                              sem_u).wait()

    def drain_c():
        pltpu.make_async_copy(cat_table.at[0], c_rows.at[0, pl.ds(0, D)],
                              sem_c).wait()

    def fire_body(g, carry):
        gb = g * L
        uv = uidx_v[pl.ds(gb, L)]
        cv = cidx_v[pl.ds(gb, L)]
        for jj in range(L):
            rr = g * (L // 2) + jj // 2
            cs = pl.ds((jj % 2) * D, D)
            pltpu.async_copy(user_table.at[uv[jj]], u_rows.at[rr, cs],
                             sem_u)
            pltpu.async_copy(cat_table.at[cv[jj]], c_rows.at[rr, cs],
                             sem_c)

        @pl.when(g >= 1)
        def _():
            for _ in range(L):
                drain_u()
                drain_c()

        return carry

    lax.fori_loop(0, BPW // L, fire_body, 0)
    for _ in range(L):
        drain_u()
        drain_c()

    wu = [w_v[pl.ds(k * L, L)] for k in range(D // L)]
    wc = [w_v[pl.ds(D + k * L, L)] for k in range(D // L)]
    lanes = lax.iota(jnp.int32, L)

    def group_body(g, carry):
        gb = g * L
        res = jnp.zeros((L,), jnp.float32)
        for jj in range(L):
            rr = g * (L // 2) + jj // 2
            cb = (jj % 2) * D
            acc = u_rows[rr, pl.ds(cb, L)] * wu[0]
            for k in range(1, D // L):
                acc = acc + u_rows[rr, pl.ds(cb + k * L, L)] * wu[k]
            for k in range(D // L):
                acc = acc + c_rows[rr, pl.ds(cb + k * L, L)] * wc[k]
            for s in (8, 4, 2, 1):
                acc = acc + _perm(acc, lanes ^ s)
            res = jnp.where(lanes == jj, acc, res)
        out_v[pl.ds(gb, L)] = res
        return carry

    lax.fori_loop(0, BPW // L, group_body, 0)

    bv = b_v[...]
    for j in range(BPW // L):
        x = out_v[pl.ds(j * L, L)] + bv
        out_v[pl.ds(j * L, L)] = 1.0 / (1.0 + jnp.exp(-x))

    pltpu.sync_copy(out_v, out_hbm.at[pl.ds(base, BPW)])


def kernel(user, category, user_table, category_table, fc_w, fc_b):
    w_flat = fc_w.reshape(2 * D)
    b_vec = jnp.broadcast_to(fc_b.reshape(1), (L,))

    run = pl.kernel(
        _sc_body,
        out_type=jax.ShapeDtypeStruct((B,), jnp.float32),
        mesh=plsc.VectorSubcoreMesh(core_axis_name="c", subcore_axis_name="s"),
        scratch_types=[
            pltpu.VMEM((BPW,), jnp.int32),
            pltpu.VMEM((BPW,), jnp.int32),
            pltpu.VMEM((BPW // 2, 2 * D), jnp.float32),
            pltpu.VMEM((BPW // 2, 2 * D), jnp.float32),
            pltpu.VMEM((BPW,), jnp.float32),
            pltpu.VMEM((2 * D,), jnp.float32),
            pltpu.VMEM((L,), jnp.float32),
            pltpu.SemaphoreType.DMA,
            pltpu.SemaphoreType.DMA,
        ],
    )
    return run(user, category, user_table, category_table, w_flat, b_vec)


# speedup vs baseline: 3.0362x; 3.0362x over previous
import jax
import jax.numpy as jnp
from jax import lax
from jax.experimental import pallas as pl
from jax.experimental.pallas import tpu as pltpu
from jax.experimental.pallas import tpu_sc as plsc

D = 64
B = 16384
L = 16
NC, NS = 2, 16
NW = NC * NS
BPW = B // NW
CHUNK = 128
NCHUNK = BPW // CHUNK
CBLK = 16384


def _score_body(w_ref, t_ref, o_ref):
    x = t_ref[...]
    w = w_ref[...]
    y = jnp.sum(x * w, axis=0)
    o_ref[...] = y.reshape(CBLK // 128, 128)


def _scores(table_t, w_col):
    n = table_t.shape[1]
    grid = (n + CBLK - 1) // CBLK
    return pl.pallas_call(
        _score_body,
        grid=(grid,),
        in_specs=[
            pl.BlockSpec((D, 1), lambda i: (0, 0)),
            pl.BlockSpec((D, CBLK), lambda i: (0, i)),
        ],
        out_specs=pl.BlockSpec((CBLK // 128, 128), lambda i: (i, 0)),
        out_shape=jax.ShapeDtypeStruct((grid * (CBLK // 128), 128),
                                       jnp.float32),
    )(w_col, table_t)


def _lookup_body(user_idx, cat_idx, su, sc, b_vec, out_hbm,
                 uidx_v, cidx_v, pu_v, pc_v, su_buf, sc_buf, out_v, b_v,
                 sem_u, sem_c):
    wid = lax.axis_index("s") * NC + lax.axis_index("c")
    base = wid * BPW

    pltpu.sync_copy(user_idx.at[pl.ds(base, BPW)], uidx_v)
    pltpu.sync_copy(cat_idx.at[pl.ds(base, BPW)], cidx_v)
    pltpu.sync_copy(b_vec, b_v)

    def prep_body(g, carry):
        gb = g * L
        uv = uidx_v[pl.ds(gb, L)]
        cv = cidx_v[pl.ds(gb, L)]
        pu_v[pl.ds(gb, L)] = lax.shift_right_logical(uv, 7)
        pc_v[pl.ds(gb, L)] = lax.shift_right_logical(cv, 7)
        return carry

    lax.fori_loop(0, BPW // L, prep_body, 0)

    lanes = lax.iota(jnp.int32, L)
    bv = b_v[...]

    for c in range(NCHUNK):
        hu = pltpu.async_copy(su.at[pu_v.at[pl.ds(c * CHUNK, CHUNK)]],
                              su_buf, sem_u)
        hc = pltpu.async_copy(sc.at[pc_v.at[pl.ds(c * CHUNK, CHUNK)]],
                              sc_buf, sem_c)
        hu.wait()
        hc.wait()

        def grp_body(g, carry):
            gb = g * L
            uv = uidx_v[pl.ds(c * CHUNK + gb, L)]
            cv = cidx_v[pl.ds(c * CHUNK + gb, L)]
            rows = gb + lanes
            vu = plsc.load_gather(su_buf, [rows, uv & 127])
            vc = plsc.load_gather(sc_buf, [rows, cv & 127])
            x = vu + vc + bv
            out_v[pl.ds(c * CHUNK + gb, L)] = 1.0 / (1.0 + jnp.exp(-x))
            return carry

        lax.fori_loop(0, CHUNK // L, grp_body, 0)

    pltpu.sync_copy(out_v, out_hbm.at[pl.ds(base, BPW)])


def kernel(user, category, user_table, category_table, fc_w, fc_b):
    ut_t = user_table.T
    ct_t = category_table.T
    wu_col = fc_w[0, :D].reshape(D, 1)
    wc_col = fc_w[0, D:].reshape(D, 1)
    b_vec = jnp.broadcast_to(fc_b.reshape(1), (L,))

    s_u = _scores(ut_t, wu_col)
    s_c = _scores(ct_t, wc_col)

    run = pl.kernel(
        _lookup_body,
        out_type=jax.ShapeDtypeStruct((B,), jnp.float32),
        mesh=plsc.VectorSubcoreMesh(core_axis_name="c", subcore_axis_name="s"),
        compiler_params=pltpu.CompilerParams(needs_layout_passes=False),
        scratch_types=[
            pltpu.VMEM((BPW,), jnp.int32),
            pltpu.VMEM((BPW,), jnp.int32),
            pltpu.VMEM((BPW,), jnp.int32),
            pltpu.VMEM((BPW,), jnp.int32),
            pltpu.VMEM((CHUNK, 128), jnp.float32),
            pltpu.VMEM((CHUNK, 128), jnp.float32),
            pltpu.VMEM((BPW,), jnp.float32),
            pltpu.VMEM((L,), jnp.float32),
            pltpu.SemaphoreType.DMA,
            pltpu.SemaphoreType.DMA,
        ],
    )
    return run(user, category, s_u, s_c, b_vec)


# speedup vs baseline: 3.3755x; 1.1118x over previous
import jax
import jax.numpy as jnp
from jax import lax
from jax.experimental import pallas as pl
from jax.experimental.pallas import tpu as pltpu
from jax.experimental.pallas import tpu_sc as plsc

D = 64
B = 16384
L = 16
NC, NS = 2, 16
NW = NC * NS
BPW = B // NW
CHUNK = 128
NCHUNK = BPW // CHUNK
CBLK = 32768


def _score_body(w_ref, t_ref, o_ref):
    x = t_ref[...]
    w = w_ref[...]
    y = jnp.sum(x * w, axis=0)
    o_ref[...] = y.reshape(CBLK // 128, 128)


def _scores(table_t, w_col):
    n = table_t.shape[1]
    grid = (n + CBLK - 1) // CBLK
    return pl.pallas_call(
        _score_body,
        grid=(grid,),
        in_specs=[
            pl.BlockSpec((D, 1), lambda i: (0, 0)),
            pl.BlockSpec((D, CBLK), lambda i: (0, i)),
        ],
        out_specs=pl.BlockSpec((CBLK // 128, 128), lambda i: (i, 0)),
        out_shape=jax.ShapeDtypeStruct((grid * (CBLK // 128), 128),
                                       jnp.float32),
    )(w_col, table_t)


def _lookup_body(user_idx, cat_idx, su, sc, b_vec, out_hbm,
                 uidx_v, cidx_v, pu_v, pc_v, su_buf, sc_buf, out_v, b_v,
                 sem_u, sem_c):
    wid = lax.axis_index("s") * NC + lax.axis_index("c")
    base = wid * BPW

    pltpu.sync_copy(user_idx.at[pl.ds(base, BPW)], uidx_v)
    pltpu.sync_copy(cat_idx.at[pl.ds(base, BPW)], cidx_v)
    pltpu.sync_copy(b_vec, b_v)

    def prep_body(g, carry):
        gb = g * L
        uv = uidx_v[pl.ds(gb, L)]
        cv = cidx_v[pl.ds(gb, L)]
        pu_v[pl.ds(gb, L)] = lax.shift_right_logical(uv, 7)
        pc_v[pl.ds(gb, L)] = lax.shift_right_logical(cv, 7)
        return carry

    lax.fori_loop(0, BPW // L, prep_body, 0)

    lanes = lax.iota(jnp.int32, L)
    bv = b_v[...]

    for c in range(NCHUNK):
        hu = pltpu.async_copy(su.at[pu_v.at[pl.ds(c * CHUNK, CHUNK)]],
                              su_buf, sem_u)
        hc = pltpu.async_copy(sc.at[pc_v.at[pl.ds(c * CHUNK, CHUNK)]],
                              sc_buf, sem_c)
        hu.wait()
        hc.wait()

        def grp_body(g, carry):
            gb = g * L
            uv = uidx_v[pl.ds(c * CHUNK + gb, L)]
            cv = cidx_v[pl.ds(c * CHUNK + gb, L)]
            rows = gb + lanes
            vu = plsc.load_gather(su_buf, [rows, uv & 127])
            vc = plsc.load_gather(sc_buf, [rows, cv & 127])
            x = vu + vc + bv
            out_v[pl.ds(c * CHUNK + gb, L)] = 1.0 / (1.0 + jnp.exp(-x))
            return carry

        lax.fori_loop(0, CHUNK // L, grp_body, 0)

    pltpu.sync_copy(out_v, out_hbm.at[pl.ds(base, BPW)])


def kernel(user, category, user_table, category_table, fc_w, fc_b):
    ut_t = user_table.T
    ct_t = category_table.T
    wu_col = fc_w[0, :D].reshape(D, 1)
    wc_col = fc_w[0, D:].reshape(D, 1)
    b_vec = jnp.broadcast_to(fc_b.reshape(1), (L,))

    s_u = _scores(ut_t, wu_col)
    s_c = _scores(ct_t, wc_col)

    run = pl.kernel(
        _lookup_body,
        out_type=jax.ShapeDtypeStruct((B,), jnp.float32),
        mesh=plsc.VectorSubcoreMesh(core_axis_name="c", subcore_axis_name="s"),
        compiler_params=pltpu.CompilerParams(needs_layout_passes=False),
        scratch_types=[
            pltpu.VMEM((BPW,), jnp.int32),
            pltpu.VMEM((BPW,), jnp.int32),
            pltpu.VMEM((BPW,), jnp.int32),
            pltpu.VMEM((BPW,), jnp.int32),
            pltpu.VMEM((CHUNK, 128), jnp.float32),
            pltpu.VMEM((CHUNK, 128), jnp.float32),
            pltpu.VMEM((BPW,), jnp.float32),
            pltpu.VMEM((L,), jnp.float32),
            pltpu.SemaphoreType.DMA,
            pltpu.SemaphoreType.DMA,
        ],
    )
    return run(user, category, s_u, s_c, b_vec)
